# baseline (speedup 1.0000x reference)
import functools

import jax
import jax.numpy as jnp
from jax import lax
from jax.experimental import pallas as pl
from jax.experimental.pallas import tpu as pltpu
from jax.experimental.pallas import tpu_sc as plsc

N_TAB_ = 10000
NR_ = 5
EMB_ = 32
B_ = 1024
SPLIT_ = 2
SUBCHK_ = 1000
CHK_ = SPLIT_ * SUBCHK_
NK_ = N_TAB_ // CHK_

SC_NC_ = 2
SC_NS_ = 16
SC_NW_ = SC_NC_ * SC_NS_
GD_ = 128


def _dot_t(a, w):
    return jax.lax.dot_general(a, w, (((1,), (1,)), ((), ())),
                               preferred_element_type=jnp.float32)


def _dot_tl(et, tab):
    return jax.lax.dot_general(et, tab, (((0,), (0,)), ((), ())),
                               preferred_element_type=jnp.float32)


def _sc_gather(tab, idx):
    b_per_w = B_ // SC_NW_
    mesh = plsc.VectorSubcoreMesh(core_axis_name="c", subcore_axis_name="s")

    @functools.partial(
        pl.kernel, mesh=mesh,
        out_type=jax.ShapeDtypeStruct((B_, GD_), jnp.float32),
        scratch_types=[
            pltpu.VMEM((b_per_w,), jnp.int32),
            pltpu.VMEM((b_per_w, GD_), jnp.float32),
            pltpu.SemaphoreType.DMA,
        ],
    )
    def k(tab_hbm, idx_hbm, out_hbm, idx_v, rows_v, sem):
        wid = lax.axis_index("s") * SC_NC_ + lax.axis_index("c")
        base = wid * b_per_w
        pltpu.sync_copy(idx_hbm.at[pl.ds(base, b_per_w)], idx_v)
        pltpu.async_copy(tab_hbm.at[idx_v], rows_v, sem).wait()
        pltpu.sync_copy(rows_v, out_hbm.at[pl.ds(base, b_per_w)])

    return k(tab, idx)


def _gcmc_body(*refs):
    (ug_ref, ig_ref, *edge_refs, utab_ref, itab_ref,
     guW_ref, gub_ref, giW_ref, gib_ref,
     f1uW_ref, f1ub_ref, f1iW_ref, f1ib_ref,
     l1W_ref, l1b_ref, l2W_ref, l2b_ref, l3W_ref, l3b_ref,
     out_ref, au_scr, ai_scr) = refs
    eUIT_refs = edge_refs[:SPLIT_]
    eIUT_refs = edge_refs[SPLIT_:]
    n = pl.program_id(0)
    k = pl.program_id(1)

    hu = hi = None
    for s in range(SPLIT_):
        off = pl.ds(k * CHK_ + s * SUBCHK_, SUBCHK_)
        pu = _dot_tl(eIUT_refs[s][0], utab_ref[off, :])
        pi = _dot_tl(eUIT_refs[s][0], itab_ref[off, :])
        hu = pu if hu is None else hu + pu
        hi = pi if hi is None else hi + pi

    @pl.when(k == 0)
    def _():
        au_scr[n] = hu
        ai_scr[n] = hi

    @pl.when(k != 0)
    def _():
        au_scr[n] += hu
        ai_scr[n] += hi

    @pl.when((n == NR_ - 1) & (k == NK_ - 1))
    def _epilogue():
        gu_h = jnp.concatenate(
            [jnp.maximum(_dot_t(au_scr[m], guW_ref[...])
                         + gub_ref[...], 0.0) for m in range(NR_)], axis=1)
        gi_h = jnp.concatenate(
            [jnp.maximum(_dot_t(ai_scr[m], giW_ref[...])
                         + gib_ref[...], 0.0) for m in range(NR_)], axis=1)
        guo = _dot_t(gu_h, f1uW_ref[...]) + f1ub_ref[...]
        gio = _dot_t(gi_h, f1iW_ref[...]) + f1ib_ref[...]

        ue_g = ug_ref[:, :EMB_]
        ub_g = ug_ref[:, EMB_:EMB_ + 1]
        ie_g = ig_ref[:, :EMB_]
        ib_g = ig_ref[:, EMB_:EMB_ + 1]

        h = jnp.concatenate(
            [ue_g * ie_g, ue_g * gio, guo * ie_g, guo * gio], axis=1)
        x1 = jnp.maximum(_dot_t(h, l1W_ref[...]) + l1b_ref[...], 0.0)
        x2 = jnp.maximum(_dot_t(x1, l2W_ref[...]) + l2b_ref[...], 0.0)
        x3 = jnp.sum(x2 * l3W_ref[...], axis=1, keepdims=True)
        x3 = x3 + l3b_ref[0, 0]
        out_ref[...] = x3 + ub_g + ib_g


def kernel(x, edge_UI, edge_IU, user_embedding, item_embedding,
           GCN_user_W, GCN_user_b, GCN_item_W, GCN_item_b,
           fc1_user_W, fc1_user_b, fc1_item_W, fc1_item_b,
           l1_W, l1_b, l2_W, l2_b, l3_W, l3_b,
           user_bias, item_bias):
    full = lambda a: pl.BlockSpec(a.shape, lambda n, k: (0,) * a.ndim)
    row2 = lambda v: v.reshape(1, -1)

    pad = jnp.zeros((N_TAB_, GD_ - EMB_ - 1), jnp.float32)
    utab48 = jnp.concatenate([user_embedding, user_bias, pad], axis=1)
    itab48 = jnp.concatenate([item_embedding, item_bias, pad], axis=1)
    ug = _sc_gather(utab48, x[:, 0])
    ig = _sc_gather(itab48, x[:, 1])

    eUIT = jnp.swapaxes(edge_UI, 1, 2)
    eIUT = jnp.swapaxes(edge_IU, 1, 2)

    def edge_spec(s):
        return pl.BlockSpec(
            (1, SUBCHK_, B_),
            lambda n, k, s=s: (n, k * SPLIT_ + s, 0))

    out = pl.pallas_call(
        _gcmc_body,
        grid=(NR_, NK_),
        in_specs=(
            [full(ug), full(ig)]
            + [edge_spec(s) for s in range(SPLIT_)]
            + [edge_spec(s) for s in range(SPLIT_)]
            + [full(user_embedding), full(item_embedding),
               full(GCN_user_W), full(row2(GCN_user_b)),
               full(GCN_item_W), full(row2(GCN_item_b)),
               full(fc1_user_W), full(row2(fc1_user_b)),
               full(fc1_item_W), full(row2(fc1_item_b)),
               full(l1_W), full(row2(l1_b)),
               full(l2_W), full(row2(l2_b)),
               full(l3_W), full(row2(l3_b))]),
        out_specs=pl.BlockSpec((B_, 1), lambda n, k: (0, 0)),
        out_shape=jax.ShapeDtypeStruct((B_, 1), jnp.float32),
        scratch_shapes=[
            pltpu.VMEM((NR_, B_, EMB_), jnp.float32),
            pltpu.VMEM((NR_, B_, EMB_), jnp.float32),
        ],
        compiler_params=pltpu.CompilerParams(
            dimension_semantics=("arbitrary", "arbitrary")),
    )(ug, ig, *([eUIT] * SPLIT_), *([eIUT] * SPLIT_),
      user_embedding, item_embedding,
      GCN_user_W, row2(GCN_user_b), GCN_item_W, row2(GCN_item_b),
      fc1_user_W, row2(fc1_user_b), fc1_item_W, row2(fc1_item_b),
      l1_W, row2(l1_b), l2_W, row2(l2_b), l3_W, row2(l3_b))
    return out.reshape(-1)


# speedup vs baseline: 1.0262x; 1.0262x over previous
import functools

import jax
import jax.numpy as jnp
from jax import lax
from jax.experimental import pallas as pl
from jax.experimental.pallas import tpu as pltpu
from jax.experimental.pallas import tpu_sc as plsc

N_TAB_ = 10000
NR_ = 5
EMB_ = 32
B_ = 1024
SPLIT_ = 2
SUBCHK_ = 1000
CHK_ = SPLIT_ * SUBCHK_
NK_ = N_TAB_ // CHK_

SC_NC_ = 2
SC_NS_ = 16
SC_NW_ = SC_NC_ * SC_NS_
GD_ = 128


def _dot_t(a, w):
    return jax.lax.dot_general(a, w, (((1,), (1,)), ((), ())),
                               preferred_element_type=jnp.float32)


def _dot_tl(et, tab):
    return jax.lax.dot_general(et, tab, (((0,), (0,)), ((), ())),
                               preferred_element_type=jnp.float32)


def _sc_gather2(utab, itab, uidx, iidx):
    b_per_w = B_ // SC_NW_
    mesh = plsc.VectorSubcoreMesh(core_axis_name="c", subcore_axis_name="s")

    @functools.partial(
        pl.kernel, mesh=mesh,
        out_type=(jax.ShapeDtypeStruct((B_, GD_), jnp.float32),
                  jax.ShapeDtypeStruct((B_, GD_), jnp.float32)),
        scratch_types=[
            pltpu.VMEM((b_per_w,), jnp.int32),
            pltpu.VMEM((b_per_w, GD_), jnp.float32),
            pltpu.SemaphoreType.DMA,
        ],
    )
    def k(utab_hbm, itab_hbm, uidx_hbm, iidx_hbm, uout_hbm, iout_hbm,
          idx_v, rows_v, sem):
        wid = lax.axis_index("s") * SC_NC_ + lax.axis_index("c")
        base = wid * b_per_w
        for tab_hbm, idx_hbm, out_hbm in (
                (utab_hbm, uidx_hbm, uout_hbm),
                (itab_hbm, iidx_hbm, iout_hbm)):
            pltpu.sync_copy(idx_hbm.at[pl.ds(base, b_per_w)], idx_v)
            pltpu.async_copy(tab_hbm.at[idx_v], rows_v, sem).wait()
            pltpu.sync_copy(rows_v, out_hbm.at[pl.ds(base, b_per_w)])

    return k(utab, itab, uidx, iidx)


def _stream_body(*refs):
    (*edge_refs, utab_ref, itab_ref, au_ref, ai_ref) = refs
    eUIT_refs = edge_refs[:SPLIT_]
    eIUT_refs = edge_refs[SPLIT_:]
    k = pl.program_id(1)

    hu = hi = None
    for s in range(SPLIT_):
        off = pl.ds(k * CHK_ + s * SUBCHK_, SUBCHK_)
        pu = _dot_tl(eIUT_refs[s][0], utab_ref[off, :])
        pi = _dot_tl(eUIT_refs[s][0], itab_ref[off, :])
        hu = pu if hu is None else hu + pu
        hi = pi if hi is None else hi + pi

    @pl.when(k == 0)
    def _():
        au_ref[0] = hu
        ai_ref[0] = hi

    @pl.when(k != 0)
    def _():
        au_ref[0] += hu
        ai_ref[0] += hi


def _epilogue_body(au_ref, ai_ref, ug_ref, ig_ref,
                   guW_ref, gub_ref, giW_ref, gib_ref,
                   f1uW_ref, f1ub_ref, f1iW_ref, f1ib_ref,
                   l1W_ref, l1b_ref, l2W_ref, l2b_ref, l3W_ref, l3b_ref,
                   out_ref):
    gu_h = jnp.concatenate(
        [jnp.maximum(_dot_t(au_ref[m], guW_ref[...])
                     + gub_ref[...], 0.0) for m in range(NR_)], axis=1)
    gi_h = jnp.concatenate(
        [jnp.maximum(_dot_t(ai_ref[m], giW_ref[...])
                     + gib_ref[...], 0.0) for m in range(NR_)], axis=1)
    guo = _dot_t(gu_h, f1uW_ref[...]) + f1ub_ref[...]
    gio = _dot_t(gi_h, f1iW_ref[...]) + f1ib_ref[...]

    ue_g = ug_ref[:, :EMB_]
    ub_g = ug_ref[:, EMB_:EMB_ + 1]
    ie_g = ig_ref[:, :EMB_]
    ib_g = ig_ref[:, EMB_:EMB_ + 1]

    h = jnp.concatenate(
        [ue_g * ie_g, ue_g * gio, guo * ie_g, guo * gio], axis=1)
    x1 = jnp.maximum(_dot_t(h, l1W_ref[...]) + l1b_ref[...], 0.0)
    x2 = jnp.maximum(_dot_t(x1, l2W_ref[...]) + l2b_ref[...], 0.0)
    x3 = jnp.sum(x2 * l3W_ref[...], axis=1, keepdims=True)
    x3 = x3 + l3b_ref[0, 0]
    out_ref[...] = x3 + ub_g + ib_g


def kernel(x, edge_UI, edge_IU, user_embedding, item_embedding,
           GCN_user_W, GCN_user_b, GCN_item_W, GCN_item_b,
           fc1_user_W, fc1_user_b, fc1_item_W, fc1_item_b,
           l1_W, l1_b, l2_W, l2_b, l3_W, l3_b,
           user_bias, item_bias):
    row2 = lambda v: v.reshape(1, -1)

    pad = jnp.zeros((N_TAB_, GD_ - EMB_ - 1), jnp.float32)
    utab128 = jnp.concatenate([user_embedding, user_bias, pad], axis=1)
    itab128 = jnp.concatenate([item_embedding, item_bias, pad], axis=1)
    ug, ig = _sc_gather2(utab128, itab128, x[:, 0], x[:, 1])

    eUIT = jnp.swapaxes(edge_UI, 1, 2)
    eIUT = jnp.swapaxes(edge_IU, 1, 2)

    def edge_spec(s):
        return pl.BlockSpec(
            (1, SUBCHK_, B_),
            lambda n, k, s=s: (n, k * SPLIT_ + s, 0))

    acc_spec = pl.BlockSpec((1, B_, EMB_), lambda n, k: (n, 0, 0))
    acc_type = jax.ShapeDtypeStruct((NR_, B_, EMB_), jnp.float32)
    full = lambda a: pl.BlockSpec(a.shape, lambda n, k: (0,) * a.ndim)

    au, ai = pl.pallas_call(
        _stream_body,
        grid=(NR_, NK_),
        in_specs=(
            [edge_spec(s) for s in range(SPLIT_)]
            + [edge_spec(s) for s in range(SPLIT_)]
            + [full(user_embedding), full(item_embedding)]),
        out_specs=(acc_spec, acc_spec),
        out_shape=(acc_type, acc_type),
        compiler_params=pltpu.CompilerParams(
            dimension_semantics=("arbitrary", "arbitrary")),
    )(*([eUIT] * SPLIT_), *([eIUT] * SPLIT_),
      user_embedding, item_embedding)

    fullg = lambda a: pl.BlockSpec(a.shape, lambda: (0,) * a.ndim)
    out = pl.pallas_call(
        _epilogue_body,
        in_specs=[fullg(a) for a in (
            au, ai, ug, ig,
            GCN_user_W, row2(GCN_user_b), GCN_item_W, row2(GCN_item_b),
            fc1_user_W, row2(fc1_user_b), fc1_item_W, row2(fc1_item_b),
            l1_W, row2(l1_b), l2_W, row2(l2_b), l3_W, row2(l3_b))],
        out_specs=pl.BlockSpec((B_, 1), lambda: (0, 0)),
        out_shape=jax.ShapeDtypeStruct((B_, 1), jnp.float32),
    )(au, ai, ug, ig,
      GCN_user_W, row2(GCN_user_b), GCN_item_W, row2(GCN_item_b),
      fc1_user_W, row2(fc1_user_b), fc1_item_W, row2(fc1_item_b),
      l1_W, row2(l1_b), l2_W, row2(l2_b), l3_W, row2(l3_b))
    return out.reshape(-1)


# speedup vs baseline: 1.0418x; 1.0152x over previous
import functools

import jax
import jax.numpy as jnp
from jax import lax
from jax.experimental import pallas as pl
from jax.experimental.pallas import tpu as pltpu
from jax.experimental.pallas import tpu_sc as plsc

N_TAB_ = 10000
NR_ = 5
EMB_ = 32
B_ = 1024
SPLIT_ = 2
SUBCHK_ = 1000
CHK_ = SPLIT_ * SUBCHK_
NK_ = N_TAB_ // CHK_

SC_NC_ = 2
SC_NS_ = 16
SC_NW_ = SC_NC_ * SC_NS_
GD_ = 128


def _dot_t(a, w):
    return jax.lax.dot_general(a, w, (((1,), (1,)), ((), ())),
                               preferred_element_type=jnp.float32)


def _dot_tl(et, tab):
    return jax.lax.dot_general(et, tab, (((0,), (0,)), ((), ())),
                               preferred_element_type=jnp.float32)


def _sc_gather4(tabs, idxs):
    b_per_w = B_ // SC_NW_
    mesh = plsc.VectorSubcoreMesh(core_axis_name="c", subcore_axis_name="s")

    @functools.partial(
        pl.kernel, mesh=mesh,
        out_type=tuple(jax.ShapeDtypeStruct((B_, GD_), jnp.float32)
                       for _ in range(4)),
        scratch_types=[
            pltpu.VMEM((b_per_w,), jnp.int32),
            pltpu.VMEM((b_per_w, GD_), jnp.float32),
            pltpu.SemaphoreType.DMA,
        ],
    )
    def k(*refs):
        tab_hbms = refs[:4]
        idx_hbms = refs[4:8]
        out_hbms = refs[8:12]
        idx_v, rows_v, sem = refs[12:]
        wid = lax.axis_index("s") * SC_NC_ + lax.axis_index("c")
        base = wid * b_per_w
        for tab_hbm, idx_hbm, out_hbm in zip(tab_hbms, idx_hbms, out_hbms):
            pltpu.sync_copy(idx_hbm.at[pl.ds(base, b_per_w)], idx_v)
            pltpu.async_copy(tab_hbm.at[idx_v], rows_v, sem).wait()
            pltpu.sync_copy(rows_v, out_hbm.at[pl.ds(base, b_per_w)])

    return k(*tabs, *idxs)


def _stream_body(*refs):
    (*edge_refs, utab_ref, itab_ref, au_ref, ai_ref) = refs
    eUIT_refs = edge_refs[:SPLIT_]
    eIUT_refs = edge_refs[SPLIT_:]
    k = pl.program_id(1)

    hu = hi = None
    for s in range(SPLIT_):
        off = pl.ds(k * CHK_ + s * SUBCHK_, SUBCHK_)
        pu = _dot_tl(eIUT_refs[s][0], utab_ref[off, :])
        pi = _dot_tl(eUIT_refs[s][0], itab_ref[off, :])
        hu = pu if hu is None else hu + pu
        hi = pi if hi is None else hi + pi

    @pl.when(k == 0)
    def _():
        au_ref[0] = hu
        ai_ref[0] = hi

    @pl.when(k != 0)
    def _():
        au_ref[0] += hu
        ai_ref[0] += hi


def _select_sub(rows, idx):
    sub = lax.bitwise_and(idx, 3)
    out = jnp.zeros((B_, EMB_), jnp.float32)
    for j in range(4):
        out = out + jnp.where(sub == j, rows[:, j * EMB_:(j + 1) * EMB_],
                              0.0)
    return out


def _select_col(rows, idx):
    col = lax.bitwise_and(idx, 127)
    lane = lax.broadcasted_iota(jnp.int32, (B_, GD_), 1)
    return jnp.sum(jnp.where(lane == col, rows, 0.0), axis=1,
                   keepdims=True)


def _epilogue_body(au_ref, ai_ref, uer_ref, ier_ref, ubr_ref, ibr_ref,
                   uid_ref, iid_ref,
                   guW_ref, gub_ref, giW_ref, gib_ref,
                   f1uW_ref, f1ub_ref, f1iW_ref, f1ib_ref,
                   l1W_ref, l1b_ref, l2W_ref, l2b_ref, l3W_ref, l3b_ref,
                   out_ref):
    gu_h = jnp.concatenate(
        [jnp.maximum(_dot_t(au_ref[m], guW_ref[...])
                     + gub_ref[...], 0.0) for m in range(NR_)], axis=1)
    gi_h = jnp.concatenate(
        [jnp.maximum(_dot_t(ai_ref[m], giW_ref[...])
                     + gib_ref[...], 0.0) for m in range(NR_)], axis=1)
    guo = _dot_t(gu_h, f1uW_ref[...]) + f1ub_ref[...]
    gio = _dot_t(gi_h, f1iW_ref[...]) + f1ib_ref[...]

    uid = uid_ref[...]
    iid = iid_ref[...]
    ue_g = _select_sub(uer_ref[...], uid)
    ie_g = _select_sub(ier_ref[...], iid)
    ub_g = _select_col(ubr_ref[...], uid)
    ib_g = _select_col(ibr_ref[...], iid)

    h = jnp.concatenate(
        [ue_g * ie_g, ue_g * gio, guo * ie_g, guo * gio], axis=1)
    x1 = jnp.maximum(_dot_t(h, l1W_ref[...]) + l1b_ref[...], 0.0)
    x2 = jnp.maximum(_dot_t(x1, l2W_ref[...]) + l2b_ref[...], 0.0)
    x3 = jnp.sum(x2 * l3W_ref[...], axis=1, keepdims=True)
    x3 = x3 + l3b_ref[0, 0]
    out_ref[...] = x3 + ub_g + ib_g


def kernel(x, edge_UI, edge_IU, user_embedding, item_embedding,
           GCN_user_W, GCN_user_b, GCN_item_W, GCN_item_b,
           fc1_user_W, fc1_user_b, fc1_item_W, fc1_item_b,
           l1_W, l1_b, l2_W, l2_b, l3_W, l3_b,
           user_bias, item_bias):
    row2 = lambda v: v.reshape(1, -1)

    uid = x[:, 0].astype(jnp.int32)
    iid = x[:, 1].astype(jnp.int32)
    uemb_v = user_embedding.reshape(N_TAB_ * EMB_ // GD_, GD_)
    iemb_v = item_embedding.reshape(N_TAB_ * EMB_ // GD_, GD_)
    nbp = ((N_TAB_ + GD_ - 1) // GD_) * GD_
    ub_v = jnp.pad(user_bias.reshape(-1), (0, nbp - N_TAB_)
                   ).reshape(nbp // GD_, GD_)
    ib_v = jnp.pad(item_bias.reshape(-1), (0, nbp - N_TAB_)
                   ).reshape(nbp // GD_, GD_)
    uer, ier, ubr, ibr = _sc_gather4(
        (uemb_v, iemb_v, ub_v, ib_v),
        (uid >> 2, iid >> 2, uid >> 7, iid >> 7))
    uid2 = uid.reshape(B_, 1)
    iid2 = iid.reshape(B_, 1)

    eUIT = jnp.swapaxes(edge_UI, 1, 2)
    eIUT = jnp.swapaxes(edge_IU, 1, 2)

    def edge_spec(s):
        return pl.BlockSpec(
            (1, SUBCHK_, B_),
            lambda n, k, s=s: (n, k * SPLIT_ + s, 0))

    acc_spec = pl.BlockSpec((1, B_, EMB_), lambda n, k: (n, 0, 0))
    acc_type = jax.ShapeDtypeStruct((NR_, B_, EMB_), jnp.float32)
    full = lambda a: pl.BlockSpec(a.shape, lambda n, k: (0,) * a.ndim)

    au, ai = pl.pallas_call(
        _stream_body,
        grid=(NR_, NK_),
        in_specs=(
            [edge_spec(s) for s in range(SPLIT_)]
            + [edge_spec(s) for s in range(SPLIT_)]
            + [full(user_embedding), full(item_embedding)]),
        out_specs=(acc_spec, acc_spec),
        out_shape=(acc_type, acc_type),
        compiler_params=pltpu.CompilerParams(
            dimension_semantics=("arbitrary", "arbitrary")),
    )(*([eUIT] * SPLIT_), *([eIUT] * SPLIT_),
      user_embedding, item_embedding)

    fullg = lambda a: pl.BlockSpec(a.shape, lambda: (0,) * a.ndim)
    out = pl.pallas_call(
        _epilogue_body,
        in_specs=[fullg(a) for a in (
            au, ai, uer, ier, ubr, ibr, uid2, iid2,
            GCN_user_W, row2(GCN_user_b), GCN_item_W, row2(GCN_item_b),
            fc1_user_W, row2(fc1_user_b), fc1_item_W, row2(fc1_item_b),
            l1_W, row2(l1_b), l2_W, row2(l2_b), l3_W, row2(l3_b))],
        out_specs=pl.BlockSpec((B_, 1), lambda: (0, 0)),
        out_shape=jax.ShapeDtypeStruct((B_, 1), jnp.float32),
    )(au, ai, uer, ier, ubr, ibr, uid2, iid2,
      GCN_user_W, row2(GCN_user_b), GCN_item_W, row2(GCN_item_b),
      fc1_user_W, row2(fc1_user_b), fc1_item_W, row2(fc1_item_b),
      l1_W, row2(l1_b), l2_W, row2(l2_b), l3_W, row2(l3_b))
    return out.reshape(-1)
